# baseline (speedup 1.0000x reference)
import functools

import jax
import jax.numpy as jnp
from jax import lax
from jax.experimental import pallas as pl
from jax.experimental.pallas import tpu as pltpu
from jax.experimental.pallas import tpu_sc as plsc

_B = 16384
_D = 1024
_C = 1000
_L = 16
_NC = 2
_NS = 16
_NW = _NC * _NS
_RPW = _B // _NW
_CH = 16
_NCHUNK = _RPW // _CH
_G = _D // _L


_GDN = lax.GatherDimensionNumbers(
    offset_dims=(), collapsed_slice_dims=(0,), start_index_map=(0,))


def _shuffle(x, idx):
    return lax.gather(x, idx[:, None], _GDN, slice_sizes=(1,),
                      mode=lax.GatherScatterMode.PROMISE_IN_BOUNDS)


def _lane_sum(x):
    lanes = lax.iota(jnp.int32, _L)
    for s in (8, 4, 2, 1):
        x = x + _shuffle(x, lanes ^ s)
    return x


def _rsqrt_nr(x):
    i = lax.bitcast_convert_type(x, jnp.int32)
    i = 0x5F3759DF - lax.shift_right_logical(i, 1)
    y = lax.bitcast_convert_type(i, jnp.float32)
    for _ in range(3):
        y = y * (1.5 - 0.5 * x * y * y)
    return y


def _make_sc_loss():
    mesh = plsc.VectorSubcoreMesh(core_axis_name="c", subcore_axis_name="s")

    @functools.partial(
        pl.kernel,
        mesh=mesh,
        out_type=jax.ShapeDtypeStruct((_NW, _L), jnp.float32),
        scratch_types=[
            pltpu.VMEM((_CH,), jnp.int32),
            pltpu.VMEM((_CH, _D), jnp.float32),
            pltpu.VMEM((_CH, _D), jnp.float32),
            pltpu.VMEM((_CH, _D), jnp.float32),
            pltpu.VMEM((_L,), jnp.float32),
            pltpu.SemaphoreType.DMA,
        ],
    )
    def sc_loss(zi_hbm, zj_hbm, lab_hbm, anc_hbm, out_hbm,
                idx_v, zi_v, zj_v, anc_v, acc_v, sem):
        wid = lax.axis_index("s") * _NC + lax.axis_index("c")
        base = wid * _RPW

        def chunk_body(ci, acc):
            b0 = base + ci * _CH
            pltpu.sync_copy(lab_hbm.at[pl.ds(b0, _CH)], idx_v)
            gat = pltpu.async_copy(anc_hbm.at[idx_v], anc_v, sem)
            pltpu.sync_copy(zi_hbm.at[pl.ds(b0, _CH)], zi_v)
            pltpu.sync_copy(zj_hbm.at[pl.ds(b0, _CH)], zj_v)
            gat.wait()

            def row_body(r, acc):
                def g_body(g, c):
                    di, dj, ni, nj = c
                    sl = pl.ds(g * _L, _L)
                    a = anc_v[r, sl]
                    zi = zi_v[r, sl]
                    zj = zj_v[r, sl]
                    return (di + zi * a, dj + zj * a,
                            ni + zi * zi, nj + zj * zj)

                z16 = jnp.zeros((_L,), jnp.float32)
                di, dj, ni, nj = lax.fori_loop(
                    0, _G, g_body, (z16, z16, z16, z16), unroll=8)
                sdi = _lane_sum(di)
                sdj = _lane_sum(dj)
                sni = jnp.maximum(_lane_sum(ni), 1e-24)
                snj = jnp.maximum(_lane_sum(nj), 1e-24)
                cos = 0.5 * (sdi * _rsqrt_nr(sni) + sdj * _rsqrt_nr(snj))
                return acc + (1.0 - cos)

            return lax.fori_loop(0, _CH, row_body, acc)

        z16 = jnp.zeros((_L,), jnp.float32)
        total = lax.fori_loop(0, _NCHUNK, chunk_body, z16)
        acc_v[...] = total
        pltpu.sync_copy(acc_v, out_hbm.at[wid])

    return sc_loss


def kernel(z_i, z_j, z_weak, labels, anchors):
    lab = labels.astype(jnp.int32)
    parts = _make_sc_loss()(z_i, z_j, lab, anchors)
    return jnp.sum(parts[:, 0]) / _B


# speedup vs baseline: 1.8419x; 1.8419x over previous
import functools

import jax
import jax.numpy as jnp
from jax import lax
from jax.experimental import pallas as pl
from jax.experimental.pallas import tpu as pltpu
from jax.experimental.pallas import tpu_sc as plsc

_B = 16384
_D = 1024
_C = 1000
_L = 16
_NC = 2
_NS = 16
_NW = _NC * _NS
_RPW = _B // _NW
_CH = 16
_NCHUNK = _RPW // _CH
_G = _D // _L

_GDN = lax.GatherDimensionNumbers(
    offset_dims=(), collapsed_slice_dims=(0,), start_index_map=(0,))


def _shuffle(x, idx):
    return lax.gather(x, idx[:, None], _GDN, slice_sizes=(1,),
                      mode=lax.GatherScatterMode.PROMISE_IN_BOUNDS)


def _lane_sum(x):
    lanes = lax.iota(jnp.int32, _L)
    for s in (8, 4, 2, 1):
        x = x + _shuffle(x, lanes ^ s)
    return x


def _rsqrt_nr(x):
    i = lax.bitcast_convert_type(x, jnp.int32)
    i = 0x5F3759DF - lax.shift_right_logical(i, 1)
    y = lax.bitcast_convert_type(i, jnp.float32)
    for _ in range(3):
        y = y * (1.5 - 0.5 * x * y * y)
    return y


def _make_sc_loss():
    mesh = plsc.VectorSubcoreMesh(core_axis_name="c", subcore_axis_name="s")

    @functools.partial(
        pl.kernel,
        mesh=mesh,
        out_type=jax.ShapeDtypeStruct((_NW, _L), jnp.float32),
        scratch_types=[
            pltpu.VMEM((_CH,), jnp.int32),
            pltpu.VMEM((_CH,), jnp.int32),
            pltpu.VMEM((_CH, _D), jnp.float32),
            pltpu.VMEM((_CH, _D), jnp.float32),
            pltpu.VMEM((_CH, _D), jnp.float32),
            pltpu.VMEM((_CH, _D), jnp.float32),
            pltpu.VMEM((_CH, _D), jnp.float32),
            pltpu.VMEM((_CH, _D), jnp.float32),
            pltpu.VMEM((_L,), jnp.float32),
            pltpu.SemaphoreType.DMA,
            pltpu.SemaphoreType.DMA,
        ],
    )
    def sc_loss(zi_hbm, zj_hbm, lab_hbm, anc_hbm, out_hbm,
                idx0, idx1, zi0, zi1, zj0, zj1, anc0, anc1, acc_v,
                sem0, sem1):
        wid = lax.axis_index("s") * _NC + lax.axis_index("c")
        base = wid * _RPW
        bufs = ((idx0, zi0, zj0, anc0, sem0),
                (idx1, zi1, zj1, anc1, sem1))

        def start(ci, buf):
            idx_v, zi_v, zj_v, anc_v, sem = buf
            b0 = base + ci * _CH
            pltpu.sync_copy(lab_hbm.at[pl.ds(b0, _CH)], idx_v)
            pltpu.async_copy(anc_hbm.at[idx_v], anc_v, sem)
            pltpu.async_copy(zi_hbm.at[pl.ds(b0, _CH)], zi_v, sem)
            pltpu.async_copy(zj_hbm.at[pl.ds(b0, _CH)], zj_v, sem)

        def drain(ci, buf):
            idx_v, zi_v, zj_v, anc_v, sem = buf
            b0 = base + ci * _CH
            pltpu.make_async_copy(anc_hbm.at[idx_v], anc_v, sem).wait()
            pltpu.make_async_copy(zi_hbm.at[pl.ds(b0, _CH)], zi_v, sem).wait()
            pltpu.make_async_copy(zj_hbm.at[pl.ds(b0, _CH)], zj_v, sem).wait()

        def compute(buf, acc):
            _, zi_v, zj_v, anc_v, _ = buf

            def row_body(r, acc):
                def g_body(g, c):
                    di, dj, ni, nj = c
                    sl = pl.ds(g * _L, _L)
                    a = anc_v[r, sl]
                    zi = zi_v[r, sl]
                    zj = zj_v[r, sl]
                    return (di + zi * a, dj + zj * a,
                            ni + zi * zi, nj + zj * zj)

                z16 = jnp.zeros((_L,), jnp.float32)
                di, dj, ni, nj = lax.fori_loop(
                    0, _G, g_body, (z16, z16, z16, z16), unroll=16)
                sdi = _lane_sum(di)
                sdj = _lane_sum(dj)
                sni = jnp.maximum(_lane_sum(ni), 1e-24)
                snj = jnp.maximum(_lane_sum(nj), 1e-24)
                cos = 0.5 * (sdi * _rsqrt_nr(sni) + sdj * _rsqrt_nr(snj))
                return acc + (1.0 - cos)

            return lax.fori_loop(0, _CH, row_body, acc)

        start(0, bufs[0])

        def outer(ci2, acc):
            for b in (0, 1):
                ci = ci2 * 2 + b

                @pl.when(ci + 1 < _NCHUNK)
                def _():
                    start(ci + 1, bufs[1 - b])

                drain(ci, bufs[b])
                acc = compute(bufs[b], acc)
            return acc

        z16 = jnp.zeros((_L,), jnp.float32)
        total = lax.fori_loop(0, _NCHUNK // 2, outer, z16)
        acc_v[...] = total
        pltpu.sync_copy(acc_v, out_hbm.at[wid])

    return sc_loss


def kernel(z_i, z_j, z_weak, labels, anchors):
    lab = labels.astype(jnp.int32)
    parts = _make_sc_loss()(z_i, z_j, lab, anchors)
    return jnp.sum(parts[:, 0]) / _B


# speedup vs baseline: 2.7735x; 1.5057x over previous
import functools

import jax
import jax.numpy as jnp
from jax import lax
from jax.experimental import pallas as pl
from jax.experimental.pallas import tpu as pltpu
from jax.experimental.pallas import tpu_sc as plsc

_B = 16384
_D = 1024
_C = 1000
_L = 16
_NC = 2
_NS = 16
_NW = _NC * _NS
_SB = 7168
_RPW = _SB // _NW
_CH = 16
_NCHUNK = _RPW // _CH
_G = _D // _L
_TB = _B - _SB
_BB = 512

_GDN = lax.GatherDimensionNumbers(
    offset_dims=(), collapsed_slice_dims=(0,), start_index_map=(0,))


def _shuffle(x, idx):
    return lax.gather(x, idx[:, None], _GDN, slice_sizes=(1,),
                      mode=lax.GatherScatterMode.PROMISE_IN_BOUNDS)


def _lane_sum(x):
    lanes = lax.iota(jnp.int32, _L)
    for s in (8, 4, 2, 1):
        x = x + _shuffle(x, lanes ^ s)
    return x


def _rsqrt_nr(x):
    i = lax.bitcast_convert_type(x, jnp.int32)
    i = 0x5F3759DF - lax.shift_right_logical(i, 1)
    y = lax.bitcast_convert_type(i, jnp.float32)
    for _ in range(3):
        y = y * (1.5 - 0.5 * x * y * y)
    return y


def _make_sc_loss():
    mesh = plsc.VectorSubcoreMesh(core_axis_name="c", subcore_axis_name="s")

    @functools.partial(
        pl.kernel,
        mesh=mesh,
        out_type=jax.ShapeDtypeStruct((_NW, _L), jnp.float32),
        scratch_types=[
            pltpu.VMEM((_CH,), jnp.int32),
            pltpu.VMEM((_CH,), jnp.int32),
            pltpu.VMEM((_CH, _D), jnp.float32),
            pltpu.VMEM((_CH, _D), jnp.float32),
            pltpu.VMEM((_CH, _D), jnp.float32),
            pltpu.VMEM((_CH, _D), jnp.float32),
            pltpu.VMEM((_CH, _D), jnp.float32),
            pltpu.VMEM((_CH, _D), jnp.float32),
            pltpu.VMEM((_L,), jnp.float32),
            pltpu.SemaphoreType.DMA,
            pltpu.SemaphoreType.DMA,
        ],
    )
    def sc_loss(zi_hbm, zj_hbm, lab_hbm, anc_hbm, out_hbm,
                idx0, idx1, zi0, zi1, zj0, zj1, anc0, anc1, acc_v,
                sem0, sem1):
        wid = lax.axis_index("s") * _NC + lax.axis_index("c")
        base = wid * _RPW
        bufs = ((idx0, zi0, zj0, anc0, sem0),
                (idx1, zi1, zj1, anc1, sem1))

        def start(ci, buf):
            idx_v, zi_v, zj_v, anc_v, sem = buf
            b0 = base + ci * _CH
            pltpu.sync_copy(lab_hbm.at[pl.ds(b0, _CH)], idx_v)
            pltpu.async_copy(anc_hbm.at[idx_v], anc_v, sem)
            pltpu.async_copy(zi_hbm.at[pl.ds(b0, _CH)], zi_v, sem)
            pltpu.async_copy(zj_hbm.at[pl.ds(b0, _CH)], zj_v, sem)

        def drain(ci, buf):
            idx_v, zi_v, zj_v, anc_v, sem = buf
            b0 = base + ci * _CH
            pltpu.make_async_copy(anc_hbm.at[idx_v], anc_v, sem).wait()
            pltpu.make_async_copy(zi_hbm.at[pl.ds(b0, _CH)], zi_v, sem).wait()
            pltpu.make_async_copy(zj_hbm.at[pl.ds(b0, _CH)], zj_v, sem).wait()

        def compute(buf, acc):
            _, zi_v, zj_v, anc_v, _ = buf

            def row_body(r, acc):
                def g_body(g, c):
                    di, dj, ni, nj = c
                    sl = pl.ds(g * _L, _L)
                    a = anc_v[r, sl]
                    zi = zi_v[r, sl]
                    zj = zj_v[r, sl]
                    return (di + zi * a, dj + zj * a,
                            ni + zi * zi, nj + zj * zj)

                z16 = jnp.zeros((_L,), jnp.float32)
                di, dj, ni, nj = lax.fori_loop(
                    0, _G, g_body, (z16, z16, z16, z16), unroll=16)
                sdi = _lane_sum(di)
                sdj = _lane_sum(dj)
                sni = jnp.maximum(_lane_sum(ni), 1e-24)
                snj = jnp.maximum(_lane_sum(nj), 1e-24)
                cos = 0.5 * (sdi * _rsqrt_nr(sni) + sdj * _rsqrt_nr(snj))
                return acc + (1.0 - cos)

            return lax.fori_loop(0, _CH, row_body, acc)

        start(0, bufs[0])

        def outer(ci2, acc):
            for b in (0, 1):
                ci = ci2 * 2 + b

                @pl.when(ci + 1 < _NCHUNK)
                def _():
                    start(ci + 1, bufs[1 - b])

                drain(ci, bufs[b])
                acc = compute(bufs[b], acc)
            return acc

        z16 = jnp.zeros((_L,), jnp.float32)
        total = lax.fori_loop(0, _NCHUNK // 2, outer, z16)
        acc_v[...] = total
        pltpu.sync_copy(acc_v, out_hbm.at[wid])

    return sc_loss


def _tc_body(lab_ref, zi_ref, zj_ref, anc_ref, out_ref):
    g = pl.program_id(0)
    zi = zi_ref[...]
    zj = zj_ref[...]
    lab = lab_ref[0, 0, :]
    ns_i = jnp.sum(zi * zi, axis=1, keepdims=True)
    ns_j = jnp.sum(zj * zj, axis=1, keepdims=True)
    rs_i = 1.0 / jnp.maximum(jnp.sqrt(ns_i), 1e-12)
    rs_j = 1.0 / jnp.maximum(jnp.sqrt(ns_j), 1e-12)
    w = zi * rs_i + zj * rs_j
    onehot = (lab[:, None] == jax.lax.broadcasted_iota(jnp.int32, (_BB, _C), 1))
    onehot = onehot.astype(jnp.bfloat16)
    gathered = jnp.dot(onehot, anc_ref[...], preferred_element_type=jnp.float32)
    blk = jnp.sum(w * gathered).reshape(1, 1)

    @pl.when(g == 0)
    def _():
        out_ref[...] = jnp.zeros_like(out_ref)

    out_ref[...] += blk


_GOFF = _SB // _BB


def _tc_cos_sum(lab3, zi, zj, anc_bf):
    return pl.pallas_call(
        _tc_body,
        grid=(_TB // _BB,),
        in_specs=[
            pl.BlockSpec((1, 1, _BB), lambda g: (g + _GOFF, 0, 0)),
            pl.BlockSpec((_BB, _D), lambda g: (g + _GOFF, 0)),
            pl.BlockSpec((_BB, _D), lambda g: (g + _GOFF, 0)),
            pl.BlockSpec((_C, _D), lambda g: (0, 0)),
        ],
        out_specs=pl.BlockSpec((1, 1), lambda g: (0, 0)),
        out_shape=jax.ShapeDtypeStruct((1, 1), jnp.float32),
    )(lab3, zi, zj, anc_bf)


def kernel(z_i, z_j, z_weak, labels, anchors):
    lab = labels.astype(jnp.int32)
    parts = _make_sc_loss()(z_i, z_j, lab, anchors)
    lab3 = lab.reshape(_B // _BB, 1, _BB)
    anc_bf = anchors.astype(jnp.bfloat16)
    tc_cos2 = _tc_cos_sum(lab3, z_i, z_j, anc_bf)
    sc_sum = jnp.sum(parts[:, 0])
    tc_sum = _TB - 0.5 * tc_cos2[0, 0]
    return (sc_sum + tc_sum) / _B
